# baseline (speedup 1.0000x reference)
import functools

import jax
import jax.numpy as jnp
from jax import lax
from jax.experimental import pallas as pl
from jax.experimental.pallas import tpu as pltpu
from jax.experimental.pallas import tpu_sc as plsc

_IDX_MINOR = 128


def _make_lookup(B, S, V, D):
    info = plsc.get_sparse_core_info()
    ncores = info.num_cores
    nsub = info.num_subcores
    lanes = info.num_lanes
    nw = ncores * nsub
    seqs_per_w = B // nw
    spr = S // _IDX_MINOR
    n_idx = seqs_per_w * spr

    mesh = plsc.VectorSubcoreMesh(core_axis_name="c", subcore_axis_name="s")

    @functools.partial(
        pl.kernel,
        out_type=jax.ShapeDtypeStruct((B, D, S), jnp.float32),
        mesh=mesh,
        compiler_params=pltpu.CompilerParams(use_tc_tiling_on_sc=False),
        scratch_types=[
            pltpu.VMEM((n_idx * _IDX_MINOR,), jnp.int32),
            pltpu.VMEM((n_idx * _IDX_MINOR,), jnp.float32),
            pltpu.VMEM((D, S), jnp.float32),
            pltpu.SemaphoreType.DMA,
            pltpu.SemaphoreType.DMA,
        ],
    )
    def lookup(
        x_hbm, xf32_hbm, tokT_hbm, posT_hbm, out_hbm,
        idx_v, obuf_v, pos_v, gsem, wsem,
    ):
        cid = lax.axis_index("c")
        sid = lax.axis_index("s")
        wid = cid * nsub + sid
        base_seq = wid * seqs_per_w

        nw_idx = n_idx * _IDX_MINOR
        pltpu.sync_copy(x_hbm.at[pl.ds(wid * nw_idx, nw_idx)], idx_v)
        pltpu.sync_copy(posT_hbm, pos_v)

        def step(cc, carry):
            @pl.when(cc >= 1)
            def _():
                pltpu.make_async_copy(
                    xf32_hbm.at[pl.ds(0, nw_idx)], obuf_v, wsem
                ).wait()

            pltpu.async_copy(tokT_hbm.at[cc].at[idx_v], obuf_v, gsem)
            pltpu.make_async_copy(
                xf32_hbm.at[pl.ds(0, nw_idx)], obuf_v, gsem
            ).wait()

            def add_body(j, c):
                for k in range(S // lanes):
                    sl = pl.ds(k * lanes, lanes)
                    plsc.addupdate(
                        obuf_v.at[pl.ds(j * S + k * lanes, lanes)],
                        pos_v[cc, sl],
                    )
                return c

            lax.fori_loop(0, seqs_per_w, add_body, 0)

            def fire_w(j, c):
                pltpu.async_copy(
                    obuf_v.at[pl.ds(j * S, S)],
                    out_hbm.at[base_seq + j, cc],
                    wsem,
                )
                return c

            lax.fori_loop(0, seqs_per_w, fire_w, 0)
            return carry

        lax.fori_loop(0, D, step, 0)

        pltpu.make_async_copy(xf32_hbm.at[pl.ds(0, nw_idx)], obuf_v, wsem).wait()

    return lookup


def kernel(x, token_table, pos_table):
    B, S = x.shape
    V, D = token_table.shape
    xf = x.reshape(B * S).astype(jnp.int32)
    lookup = _make_lookup(B, S, V, D)
    out = lookup(
        xf,
        jax.lax.bitcast_convert_type(xf, jnp.float32),
        token_table.T,
        pos_table.T,
    )
    return out.transpose(0, 2, 1)


# speedup vs baseline: 4.1747x; 4.1747x over previous
import functools

import jax
import jax.numpy as jnp
from jax import lax
from jax.experimental import pallas as pl
from jax.experimental.pallas import tpu as pltpu
from jax.experimental.pallas import tpu_sc as plsc

_IDX_CHUNK = 128
_NBUF = 4
_UNROLL = 8


def _make_lookup(B, S, V, D):
    info = plsc.get_sparse_core_info()
    ncores = info.num_cores
    nw = ncores * info.num_subcores
    lanes = info.num_lanes
    seqs_per_w = B // nw
    n_gather = S // _IDX_CHUNK
    N = B * S

    mesh = plsc.VectorSubcoreMesh(core_axis_name="c", subcore_axis_name="s")

    @functools.partial(
        pl.kernel,
        out_type=jax.ShapeDtypeStruct((N, D), jnp.float32),
        mesh=mesh,
        compiler_params=pltpu.CompilerParams(use_tc_tiling_on_sc=False),
        scratch_types=[
            pltpu.VMEM((seqs_per_w * n_gather, _IDX_CHUNK), jnp.int32),
            [pltpu.VMEM((S, D), jnp.float32)] * _NBUF,
            pltpu.VMEM((S, D), jnp.float32),
            pltpu.VMEM((S, D), jnp.float32),
            [pltpu.SemaphoreType.DMA] * _NBUF,
            pltpu.SemaphoreType.DMA,
        ],
    )
    def lookup(
        x_hbm, tok_hbm, pos_hbm, out_hbm, idx_v, rows_v, pos_v, obuf_v, gsems, wsem
    ):
        cid = lax.axis_index("c")
        sid = lax.axis_index("s")
        wid = sid * ncores + cid
        base_seq = wid * seqs_per_w

        pltpu.sync_copy(
            x_hbm.at[pl.ds(base_seq * n_gather, seqs_per_w * n_gather)], idx_v
        )
        pltpu.sync_copy(pos_hbm, pos_v)

        def fire(tl, b):
            for j in range(n_gather):
                pltpu.async_copy(
                    tok_hbm.at[idx_v.at[tl * n_gather + j]],
                    rows_v[b].at[pl.ds(j * _IDX_CHUNK, _IDX_CHUNK)],
                    gsems[b],
                )

        def drain_g(b):
            pltpu.make_async_copy(
                tok_hbm.at[pl.ds(0, S)], rows_v[b], gsems[b]
            ).wait()

        def drain_w():
            pltpu.make_async_copy(tok_hbm.at[pl.ds(0, S)], obuf_v, wsem).wait()

        def add_pos(b):
            def body(r, c):
                r0 = r * _UNROLL
                for u in range(_UNROLL):
                    for h in range(D // lanes):
                        sl = pl.ds(h * lanes, lanes)
                        obuf_v[r0 + u, sl] = rows_v[b][r0 + u, sl] + pos_v[r0 + u, sl]
                return c

            lax.fori_loop(0, S // _UNROLL, body, 0)

        for b in range(_NBUF):
            fire(b, b)

        def group(i, c):
            for b in range(_NBUF):
                tl = i * _NBUF + b

                drain_g(b)

                @pl.when((i > 0) | (b > 0))
                def _():
                    drain_w()

                add_pos(b)
                pltpu.async_copy(
                    obuf_v, out_hbm.at[pl.ds((base_seq + tl) * S, S)], wsem
                )
                fire(lax.rem(tl + _NBUF, seqs_per_w), b)
            return c

        lax.fori_loop(0, seqs_per_w // _NBUF, group, 0)
        for b in range(_NBUF):
            drain_g(b)
        drain_w()

    return lookup


def kernel(x, token_table, pos_table):
    B, S = x.shape
    V, D = token_table.shape
    xf = x.reshape(B * S // _IDX_CHUNK, _IDX_CHUNK).astype(jnp.int32)
    lookup = _make_lookup(B, S, V, D)
    out = lookup(xf, token_table, pos_table)
    return out.reshape(B, S, D)
